# baseline (speedup 1.0000x reference)
import jax
import jax.numpy as jnp
from jax.experimental import pallas as pl
from jax.experimental.pallas import tpu as pltpu

BM = 100


def _wgcn_body(a0_ref, a1_ref, w0_ref, w1_ref, x_ref, wp_ref, b_ref, o_ref):
    aw0 = a0_ref[0] * w0_ref[0]
    aw1 = a1_ref[0] * w1_ref[0]
    agg0 = jnp.dot(aw0, x_ref[...], preferred_element_type=jnp.float32)
    agg1 = jnp.dot(aw1, x_ref[...], preferred_element_type=jnp.float32)
    p0 = jnp.dot(agg0, wp_ref[...], preferred_element_type=jnp.float32)
    p1 = jnp.dot(agg1, wp_ref[...], preferred_element_type=jnp.float32)
    o_ref[0, :BM] = jnp.maximum(p0 + b_ref[...], 0.0)
    o_ref[0, BM:] = jnp.maximum(p1 + b_ref[...], 0.0)


def kernel(adjacency_matrix, weight_matrix, data, W, b):
    n, c = data.shape
    f = W.shape[1]
    nb = n // BM
    ni = nb // 2
    a3 = adjacency_matrix.reshape(nb, BM, n)
    w3 = weight_matrix.reshape(nb, BM, n)
    b2 = b.reshape(1, f).astype(jnp.float32)

    out3 = pl.pallas_call(
        _wgcn_body,
        grid=(ni,),
        in_specs=[
            pl.BlockSpec((1, BM, n), lambda i: (2 * i, 0, 0)),
            pl.BlockSpec((1, BM, n), lambda i: (2 * i + 1, 0, 0)),
            pl.BlockSpec((1, BM, n), lambda i: (2 * i, 0, 0)),
            pl.BlockSpec((1, BM, n), lambda i: (2 * i + 1, 0, 0)),
            pl.BlockSpec((n, c), lambda i: (0, 0)),
            pl.BlockSpec((c, f), lambda i: (0, 0)),
            pl.BlockSpec((1, f), lambda i: (0, 0)),
        ],
        out_specs=pl.BlockSpec((1, 2 * BM, f), lambda i: (i, 0, 0)),
        out_shape=jax.ShapeDtypeStruct((ni, 2 * BM, f), jnp.float32),
        compiler_params=pltpu.CompilerParams(
            dimension_semantics=("parallel",),
        ),
    )(a3, a3, w3, w3, data, W, b2)
    return out3.reshape(n, f)


# speedup vs baseline: 4.0636x; 4.0636x over previous
import jax
import jax.numpy as jnp
from jax.experimental import pallas as pl
from jax.experimental.pallas import tpu as pltpu

BM = 200


def _wgcn_body(a_ref, wm_ref, x_ref, w_ref, b_ref, o_ref):
    aw = a_ref[...] * wm_ref[...]
    agg = jnp.dot(aw, x_ref[...], preferred_element_type=jnp.float32)
    proj = jnp.dot(agg, w_ref[...], preferred_element_type=jnp.float32)
    o_ref[...] = jnp.maximum(proj + b_ref[...], 0.0)


def kernel(adjacency_matrix, weight_matrix, data, W, b):
    n, c = data.shape
    f = W.shape[1]
    bm = BM if n % BM == 0 else n
    ni = n // bm
    b2 = b.reshape(1, f).astype(jnp.float32)

    return pl.pallas_call(
        _wgcn_body,
        grid=(ni,),
        in_specs=[
            pl.BlockSpec((bm, n), lambda i: (i, 0)),
            pl.BlockSpec((bm, n), lambda i: (i, 0)),
            pl.BlockSpec((n, c), lambda i: (0, 0)),
            pl.BlockSpec((c, f), lambda i: (0, 0)),
            pl.BlockSpec((1, f), lambda i: (0, 0)),
        ],
        out_specs=pl.BlockSpec((bm, f), lambda i: (i, 0)),
        out_shape=jax.ShapeDtypeStruct((n, f), jnp.float32),
        compiler_params=pltpu.CompilerParams(
            dimension_semantics=("parallel",),
        ),
    )(adjacency_matrix, weight_matrix, data, W, b2)
